# speedup vs baseline: 2.8257x; 2.8257x over previous
import jax
import jax.numpy as jnp
from jax.experimental import pallas as pl


BATCH = 4
SEQ_LEN = 4096
D_MODEL = 1024
BLOCK_S = 512


def _add_block(x_ref, pos_ref, o_ref):
    o_ref[...] = x_ref[...] + pos_ref[...][None]


def kernel(x, pos_table):
    grid = (SEQ_LEN // BLOCK_S, BATCH)
    return pl.pallas_call(
        _add_block,
        grid=grid,
        in_specs=[
            pl.BlockSpec((1, BLOCK_S, D_MODEL), lambda s, b: (b, s, 0)),
            pl.BlockSpec((BLOCK_S, D_MODEL), lambda s, b: (s, 0)),
        ],
        out_specs=pl.BlockSpec((1, BLOCK_S, D_MODEL), lambda s, b: (b, s, 0)),
        out_shape=jax.ShapeDtypeStruct((BATCH, SEQ_LEN, D_MODEL), x.dtype),
    )(x, pos_table)


# speedup vs baseline: 3.1282x; 1.1070x over previous
import jax
import jax.numpy as jnp
from jax.experimental import pallas as pl


BATCH = 4
SEQ_LEN = 4096
D_MODEL = 1024
BLOCK_S = 1024


def _add_block(x_ref, pos_ref, o_ref):
    o_ref[...] = x_ref[...] + pos_ref[...][None]


def kernel(x, pos_table):
    grid = (SEQ_LEN // BLOCK_S, BATCH)
    return pl.pallas_call(
        _add_block,
        grid=grid,
        in_specs=[
            pl.BlockSpec((1, BLOCK_S, D_MODEL), lambda s, b: (b, s, 0)),
            pl.BlockSpec((BLOCK_S, D_MODEL), lambda s, b: (s, 0)),
        ],
        out_specs=pl.BlockSpec((1, BLOCK_S, D_MODEL), lambda s, b: (b, s, 0)),
        out_shape=jax.ShapeDtypeStruct((BATCH, SEQ_LEN, D_MODEL), x.dtype),
    )(x, pos_table)


# speedup vs baseline: 3.3122x; 1.0588x over previous
import jax
import jax.numpy as jnp
from jax.experimental import pallas as pl


BATCH = 4
SEQ_LEN = 4096
D_MODEL = 1024
BLOCK_S = 2048


def _add_block(x_ref, pos_ref, o_ref):
    o_ref[...] = x_ref[...] + pos_ref[...][None]


def kernel(x, pos_table):
    grid = (SEQ_LEN // BLOCK_S, BATCH)
    return pl.pallas_call(
        _add_block,
        grid=grid,
        in_specs=[
            pl.BlockSpec((1, BLOCK_S, D_MODEL), lambda s, b: (b, s, 0)),
            pl.BlockSpec((BLOCK_S, D_MODEL), lambda s, b: (s, 0)),
        ],
        out_specs=pl.BlockSpec((1, BLOCK_S, D_MODEL), lambda s, b: (b, s, 0)),
        out_shape=jax.ShapeDtypeStruct((BATCH, SEQ_LEN, D_MODEL), x.dtype),
    )(x, pos_table)
